# baseline (speedup 1.0000x reference)
import functools

import jax
import jax.numpy as jnp
from jax import lax
from jax.experimental import pallas as pl
from jax.experimental.pallas import tpu as pltpu
from jax.experimental.pallas import tpu_sc as plsc

T = 2048
TH = T // 2
D = 768
E = 8
H = 2 * D
BT = 256
MAXT = 16
P = MAXT * BT
NW = 32
EPS = 1e-5


def _router_body(x_ref, wg_ref, pab0_ref, pab1_ref, wa_ref, wb_ref,
                 ept0_ref, ept1_ref):
    xv = x_ref[...]
    logits = jnp.dot(xv, wg_ref[...], preferred_element_type=jnp.float32)
    m = jnp.max(logits, axis=-1, keepdims=True)
    p = jnp.exp(logits - m)
    g = p / jnp.sum(p, axis=-1, keepdims=True)

    iota_e = lax.broadcasted_iota(jnp.int32, (T, E), 1)
    w1 = jnp.max(g, axis=-1, keepdims=True)
    idx1 = jnp.min(jnp.where(g == w1, iota_e, E), axis=-1, keepdims=True)
    oh1 = (iota_e == idx1)
    g2 = jnp.where(oh1, -1.0, g)
    w2 = jnp.max(g2, axis=-1, keepdims=True)
    idx2 = jnp.min(jnp.where(g2 == w2, iota_e, E), axis=-1, keepdims=True)
    oh2 = (iota_e == idx2)
    s = w1 + w2
    ones = jnp.ones((1, 128), jnp.float32)
    wa_ref[...] = (w1 / s) * ones
    wb_ref[...] = (w2 / s) * ones

    oh1f = oh1.astype(jnp.float32)
    oh2f = oh2.astype(jnp.float32)
    r = lax.broadcasted_iota(jnp.int32, (T, T), 0)
    c = lax.broadcasted_iota(jnp.int32, (T, T), 1)
    strict = (r > c).astype(jnp.bfloat16)
    excl1 = jnp.dot(strict, oh1.astype(jnp.bfloat16),
                    preferred_element_type=jnp.float32)
    excl2 = jnp.dot(strict, oh2.astype(jnp.bfloat16),
                    preferred_element_type=jnp.float32)
    c1 = jnp.sum(oh1f, axis=0, keepdims=True)
    c2 = jnp.sum(oh2f, axis=0, keepdims=True)
    c1a = jnp.sum(oh1f[:TH], axis=0, keepdims=True)
    c2a = jnp.sum(oh2f[:TH], axis=0, keepdims=True)
    c1b = c1 - c1a
    c2b = c2 - c2a

    ke = lax.broadcasted_iota(jnp.int32, (E, E), 0)
    ee = lax.broadcasted_iota(jnp.int32, (E, E), 1)
    m8 = (ke < ee).astype(jnp.float32)
    cnt0 = c1a + c2a
    cnt1 = c1b + c2b
    padc0 = jnp.ceil(cnt0 / BT) * BT
    padc1 = jnp.ceil(cnt1 / BT) * BT
    base0 = jnp.dot(padc0, m8, preferred_element_type=jnp.float32)
    base1 = jnp.dot(padc1, m8, preferred_element_type=jnp.float32)

    def sel(v, oh):
        return jnp.sum(v * oh, axis=-1, keepdims=True)

    tmask = lax.broadcasted_iota(jnp.int32, (T, 1), 0) < TH
    r1 = sel(excl1, oh1f) - jnp.where(tmask, 0.0, sel(c1a, oh1f))
    r2f = sel(excl2, oh2f)
    r2 = jnp.where(tmask, sel(c1a, oh2f) + r2f,
                   sel(c1b, oh2f) + r2f - sel(c2a, oh2f))
    pos1 = jnp.where(tmask, sel(base0, oh1f), sel(base1, oh1f)) + r1
    pos2 = jnp.where(tmask, sel(base0, oh2f), sel(base1, oh2f)) + r2
    pos1 = pos1.astype(jnp.int32).reshape(T)
    pos2 = pos2.astype(jnp.int32).reshape(T)
    pab0_ref[pl.ds(0, TH)] = pos1[:TH]
    pab0_ref[pl.ds(TH, TH)] = pos2[:TH]
    pab1_ref[pl.ds(0, TH)] = pos1[TH:]
    pab1_ref[pl.ds(TH, TH)] = pos2[TH:]

    tstart = (lax.broadcasted_iota(jnp.int32, (MAXT, E), 0) * BT)
    tstart = tstart.astype(jnp.float32)
    eid = lax.broadcasted_iota(jnp.int32, (MAXT, E), 1)
    ind0 = (tstart >= base0) & (tstart < base0 + padc0)
    ind1 = (tstart >= base1) & (tstart < base1 + padc1)
    ept0_ref[...] = jnp.sum(jnp.where(ind0, eid + 1, 0), axis=-1) - 1
    ept1_ref[...] = jnp.sum(jnp.where(ind1, eid + 1, 0), axis=-1) - 1


def _router(xf, wg):
    return pl.pallas_call(
        _router_body,
        out_shape=(
            jax.ShapeDtypeStruct((2 * TH,), jnp.int32),
            jax.ShapeDtypeStruct((2 * TH,), jnp.int32),
            jax.ShapeDtypeStruct((T, 128), jnp.float32),
            jax.ShapeDtypeStruct((T, 128), jnp.float32),
            jax.ShapeDtypeStruct((MAXT,), jnp.int32),
            jax.ShapeDtypeStruct((MAXT,), jnp.int32),
        ),
    )(xf, wg)


def _sc_mesh():
    return plsc.VectorSubcoreMesh(core_axis_name="c", subcore_axis_name="s")


def _dispatch_gather(xf, pab, hbase):
    b_per_w = P // NW
    chunk = 64
    nch = b_per_w // chunk

    @functools.partial(
        pl.kernel,
        mesh=_sc_mesh(),
        out_type=jax.ShapeDtypeStruct((P, D), jnp.float32),
        scratch_types=[
            pltpu.VMEM((2 * TH,), jnp.int32),
            pltpu.VMEM((b_per_w,), jnp.int32),
            pltpu.VMEM((chunk, D), jnp.float32),
            pltpu.VMEM((chunk, D), jnp.float32),
            pltpu.SemaphoreType.DMA,
            pltpu.SemaphoreType.DMA,
        ],
        compiler_params=pltpu.CompilerParams(needs_layout_passes=False),
        name=f"sc_dispatch_gather{hbase}",
    )
    def k(x_hbm, pab_hbm, xs_hbm, pos_v, tok_v, rows0, rows1, s0, s1):
        wid = lax.axis_index("s") * 2 + lax.axis_index("c")
        base = wid * b_per_w
        pltpu.sync_copy(pab_hbm, pos_v)
        lane = lax.iota(jnp.int32, 16)

        def init(j, carry):
            tok_v[pl.ds(j * 16, 16)] = (
                hbase + ((base + j * 16 + lane) & (TH - 1)))
            return carry

        lax.fori_loop(0, b_per_w // 16, init, 0)

        def scan(i, carry):
            p16 = pos_v[pl.ds(i * 16, 16)]
            rel = p16 - base
            msk = (rel >= 0) & (rel < b_per_w)
            t16 = hbase + ((i * 16 + lane) & (TH - 1))
            plsc.store_scatter(tok_v, [jnp.where(msk, rel, 0)], t16, mask=msk)
            return carry

        lax.fori_loop(0, (2 * TH) // 16, scan, 0)

        bufs, sems = (rows0, rows1), (s0, s1)
        cps = [None] * nch
        for j in range(nch):
            b = j % 2
            if j >= 2:
                cps[j - 2].wait()
                pltpu.sync_copy(bufs[b],
                                xs_hbm.at[pl.ds(base + (j - 2) * chunk, chunk)])
            cps[j] = pltpu.async_copy(
                x_hbm.at[tok_v.at[pl.ds(j * chunk, chunk)]], bufs[b], sems[b])
        for j in range(max(nch - 2, 0), nch):
            cps[j].wait()
            pltpu.sync_copy(bufs[j % 2],
                            xs_hbm.at[pl.ds(base + j * chunk, chunk)])

    return k(xf, pab)


def _sc_gather(table, idx, nrows, chunk, tag):
    b_per_w = nrows // NW
    nch = b_per_w // chunk
    dm = table.shape[1]
    dt = table.dtype

    @functools.partial(
        pl.kernel,
        mesh=_sc_mesh(),
        out_type=jax.ShapeDtypeStruct((nrows, dm), dt),
        scratch_types=[
            pltpu.VMEM((b_per_w,), jnp.int32),
            pltpu.VMEM((chunk, dm), dt),
            pltpu.VMEM((chunk, dm), dt),
            pltpu.SemaphoreType.DMA,
            pltpu.SemaphoreType.DMA,
        ],
        name=f"sc_gather{tag}",
    )
    def k(table_hbm, idx_hbm, out_hbm, idx_v, rows0, rows1, s0, s1):
        wid = lax.axis_index("s") * 2 + lax.axis_index("c")
        base = wid * b_per_w
        pltpu.sync_copy(idx_hbm.at[pl.ds(base, b_per_w)], idx_v)

        bufs, sems = (rows0, rows1), (s0, s1)
        cps = [None] * nch
        for j in range(nch):
            b = j % 2
            if j >= 2:
                cps[j - 2].wait()
                pltpu.sync_copy(bufs[b],
                                out_hbm.at[pl.ds(base + (j - 2) * chunk, chunk)])
            cps[j] = pltpu.async_copy(
                table_hbm.at[idx_v.at[pl.ds(j * chunk, chunk)]], bufs[b],
                sems[b])
        for j in range(max(nch - 2, 0), nch):
            cps[j].wait()
            pltpu.sync_copy(bufs[j % 2],
                            out_hbm.at[pl.ds(base + j * chunk, chunk)])

    return k(table, idx)


def _ffn_body(ept_ref, xs_ref, w1_ref, b1_ref, g1_ref, be1_ref, w2_ref,
              b2_ref, ys_ref):
    e = ept_ref[pl.program_id(0)]

    @pl.when(e >= 0)
    def _():
        h = jnp.dot(xs_ref[...].astype(jnp.bfloat16),
                    w1_ref[0].astype(jnp.bfloat16),
                    preferred_element_type=jnp.float32)
        h = h + b1_ref[0]
        mu = jnp.mean(h, axis=-1, keepdims=True)
        var = jnp.mean((h - mu) ** 2, axis=-1, keepdims=True)
        h = (h - mu) * lax.rsqrt(var + EPS) * g1_ref[0] + be1_ref[0]
        h = 0.5 * h * (1.0 + lax.erf(h * (2.0 ** -0.5)))
        y = jnp.dot(h.astype(jnp.bfloat16), w2_ref[0].astype(jnp.bfloat16),
                    preferred_element_type=jnp.float32)
        ys_ref[...] = y + b2_ref[0]


def _grouped_ffn(ept, xs, w1, b1, g1, be1, w2, b2):
    def clamp(ep, i):
        v = ep[i]
        return jnp.where(v < 0, E - 1, v)

    grid_spec = pltpu.PrefetchScalarGridSpec(
        num_scalar_prefetch=1,
        grid=(MAXT,),
        in_specs=[
            pl.BlockSpec((BT, D), lambda i, ep: (i, 0)),
            pl.BlockSpec((1, D, H), lambda i, ep: (clamp(ep, i), 0, 0)),
            pl.BlockSpec((1, 1, H), lambda i, ep: (clamp(ep, i), 0, 0)),
            pl.BlockSpec((1, 1, H), lambda i, ep: (clamp(ep, i), 0, 0)),
            pl.BlockSpec((1, 1, H), lambda i, ep: (clamp(ep, i), 0, 0)),
            pl.BlockSpec((1, H, D), lambda i, ep: (clamp(ep, i), 0, 0)),
            pl.BlockSpec((1, 1, D), lambda i, ep: (clamp(ep, i), 0, 0)),
        ],
        out_specs=pl.BlockSpec((BT, D), lambda i, ep: (i, 0)),
    )
    return pl.pallas_call(
        _ffn_body,
        grid_spec=grid_spec,
        out_shape=jax.ShapeDtypeStruct((P, D), jnp.float32),
    )(ept, xs, w1, b1.reshape(E, 1, H), g1.reshape(E, 1, H),
      be1.reshape(E, 1, H), w2, b2.reshape(E, 1, D))


def _combine_body(x_ref, ga0_ref, gb0_ref, ga1_ref, gb1_ref, wa_ref, wb_ref,
                  gf_ref, bf_ref, o_ref):
    def ln(comb):
        mu = jnp.mean(comb, axis=-1, keepdims=True)
        var = jnp.mean((comb - mu) ** 2, axis=-1, keepdims=True)
        return (comb - mu) * lax.rsqrt(var + EPS) * gf_ref[0] + bf_ref[0]

    c0 = (x_ref[0] + wa_ref[0, :, 0:1] * ga0_ref[...]
          + wb_ref[0, :, 0:1] * gb0_ref[...])
    c1 = (x_ref[1] + wa_ref[1, :, 0:1] * ga1_ref[...]
          + wb_ref[1, :, 0:1] * gb1_ref[...])
    o_ref[0] = ln(c0)
    o_ref[1] = ln(c1)


def _combine(xf, g0, g1, wa, wb, gf, bf):
    nblk = TH // BT
    return pl.pallas_call(
        _combine_body,
        grid=(nblk,),
        in_specs=[
            pl.BlockSpec((2, BT, D), lambda i: (0, i, 0)),
            pl.BlockSpec((BT, D), lambda i: (i, 0)),
            pl.BlockSpec((BT, D), lambda i: (i + nblk, 0)),
            pl.BlockSpec((BT, D), lambda i: (i, 0)),
            pl.BlockSpec((BT, D), lambda i: (i + nblk, 0)),
            pl.BlockSpec((2, BT, 128), lambda i: (0, i, 0)),
            pl.BlockSpec((2, BT, 128), lambda i: (0, i, 0)),
            pl.BlockSpec((1, D), lambda i: (0, 0)),
            pl.BlockSpec((1, D), lambda i: (0, 0)),
        ],
        out_specs=pl.BlockSpec((2, BT, D), lambda i: (0, i, 0)),
        out_shape=jax.ShapeDtypeStruct((2, TH, D), jnp.float32),
    )(xf.reshape(2, TH, D), g0, g0, g1, g1, wa.reshape(2, TH, 128),
      wb.reshape(2, TH, 128), gf, bf)


def kernel(x, Wg, W1, b1, g1, be1, W2, b2, gf, bf):
    orig_shape = x.shape
    xf = x.reshape(T, D)
    pab0, pab1, wa, wb, ept0, ept1 = _router(xf, Wg)
    xs0 = _dispatch_gather(xf, pab0, 0)
    xs1 = _dispatch_gather(xf, pab1, TH)
    ys0 = _grouped_ffn(ept0, xs0, W1, b1, g1, be1, W2, b2)
    ys1 = _grouped_ffn(ept1, xs1, W1, b1, g1, be1, W2, b2)
    g0 = _sc_gather(ys0, pab0, 2 * TH, 32, "h0")
    g1_ = _sc_gather(ys1, pab1, 2 * TH, 32, "h1")
    out = _combine(xf, g0, g1_, wa, wb, gf.reshape(1, D), bf.reshape(1, D))
    return out.reshape(orig_shape)


# speedup vs baseline: 1.1630x; 1.1630x over previous
import functools

import jax
import jax.numpy as jnp
from jax import lax
from jax.experimental import pallas as pl
from jax.experimental.pallas import tpu as pltpu
from jax.experimental.pallas import tpu_sc as plsc

T = 2048
D = 768
E = 8
H = 2 * D
BT = 256
MAXT = 24
P = MAXT * BT
NW = 32
EPS = 1e-5


def _router_body(x_ref, wg_ref, pab_ref, wa_ref, wb_ref, ept_ref,
                 oh1_s, oh2_s, ex1_s, ex2_s):
    xv = x_ref[...]
    logits = jnp.dot(xv, wg_ref[...], preferred_element_type=jnp.float32)
    m = jnp.max(logits, axis=-1, keepdims=True)
    p = jnp.exp(logits - m)
    g = p / jnp.sum(p, axis=-1, keepdims=True)

    iota_e = lax.broadcasted_iota(jnp.int32, (T, E), 1)
    w1 = jnp.max(g, axis=-1, keepdims=True)
    idx1 = jnp.min(jnp.where(g == w1, iota_e, E), axis=-1, keepdims=True)
    oh1 = (iota_e == idx1)
    g2 = jnp.where(oh1, -1.0, g)
    w2 = jnp.max(g2, axis=-1, keepdims=True)
    idx2 = jnp.min(jnp.where(g2 == w2, iota_e, E), axis=-1, keepdims=True)
    oh2 = (iota_e == idx2)
    s = w1 + w2
    wa = w1 / s
    wb = w2 / s

    oh1f = oh1.astype(jnp.float32)
    oh2f = oh2.astype(jnp.float32)
    oh1_s[...] = oh1f
    oh2_s[...] = oh2f
    ch = 128
    rr = lax.broadcasted_iota(jnp.int32, (ch, ch), 0)
    cc = lax.broadcasted_iota(jnp.int32, (ch, ch), 1)
    s128 = (rr > cc).astype(jnp.float32)

    def cs_body(j, carry):
        ca, cb = carry
        a = oh1_s[pl.ds(j * ch, ch), :]
        b = oh2_s[pl.ds(j * ch, ch), :]
        ex1_s[pl.ds(j * ch, ch), :] = (
            jnp.dot(s128, a, preferred_element_type=jnp.float32) + ca)
        ex2_s[pl.ds(j * ch, ch), :] = (
            jnp.dot(s128, b, preferred_element_type=jnp.float32) + cb)
        return (ca + jnp.sum(a, axis=0, keepdims=True),
                cb + jnp.sum(b, axis=0, keepdims=True))

    c1, c2 = lax.fori_loop(
        0, T // ch, cs_body,
        (jnp.zeros((1, E), jnp.float32), jnp.zeros((1, E), jnp.float32)))
    excl1 = ex1_s[...]
    excl2 = ex2_s[...]
    counts = c1 + c2
    padc = jnp.ceil(counts / BT) * BT
    ke = lax.broadcasted_iota(jnp.int32, (E, E), 0)
    ee = lax.broadcasted_iota(jnp.int32, (E, E), 1)
    m8 = (ke < ee).astype(jnp.float32)
    base = jnp.dot(padc, m8, preferred_element_type=jnp.float32)

    rank1 = jnp.sum(excl1 * oh1f, axis=-1, keepdims=True)
    rank2 = jnp.sum((excl2 + c1) * oh2f, axis=-1, keepdims=True)
    pos1 = jnp.sum(base * oh1f, axis=-1, keepdims=True) + rank1
    pos2 = jnp.sum(base * oh2f, axis=-1, keepdims=True) + rank2

    pab_ref[pl.ds(0, T)] = pos1.astype(jnp.int32).reshape(T)
    pab_ref[pl.ds(T, T)] = pos2.astype(jnp.int32).reshape(T)
    ones = jnp.ones((1, 128), jnp.float32)
    wa_ref[...] = wa * ones
    wb_ref[...] = wb * ones

    tstart = (lax.broadcasted_iota(jnp.int32, (MAXT, E), 0) * BT).astype(jnp.float32)
    eid = lax.broadcasted_iota(jnp.int32, (MAXT, E), 1)
    ind = (tstart >= base) & (tstart < base + padc)
    ept_ref[...] = jnp.sum(jnp.where(ind, eid + 1, 0), axis=-1).astype(jnp.int32) - 1


def _router(xf, wg):
    return pl.pallas_call(
        _router_body,
        out_shape=(
            jax.ShapeDtypeStruct((2 * T,), jnp.int32),
            jax.ShapeDtypeStruct((T, 128), jnp.float32),
            jax.ShapeDtypeStruct((T, 128), jnp.float32),
            jax.ShapeDtypeStruct((MAXT,), jnp.int32),
        ),
        scratch_shapes=[
            pltpu.VMEM((T, E), jnp.float32),
            pltpu.VMEM((T, E), jnp.float32),
            pltpu.VMEM((T, E), jnp.float32),
            pltpu.VMEM((T, E), jnp.float32),
        ],
    )(xf, wg)


def _sc_mesh():
    return plsc.VectorSubcoreMesh(core_axis_name="c", subcore_axis_name="s")


def _dispatch_gather(xf, pab):
    b_per_w = P // NW
    chunk = 48
    nch = b_per_w // chunk

    @functools.partial(
        pl.kernel,
        mesh=_sc_mesh(),
        out_type=jax.ShapeDtypeStruct((P, D), jnp.float32),
        scratch_types=[
            pltpu.VMEM((2 * T,), jnp.int32),
            pltpu.VMEM((b_per_w,), jnp.int32),
            pltpu.VMEM((chunk, D), jnp.float32),
            pltpu.VMEM((chunk, D), jnp.float32),
            pltpu.SemaphoreType.DMA,
            pltpu.SemaphoreType.DMA,
        ],
        compiler_params=pltpu.CompilerParams(needs_layout_passes=False),
        name="sc_dispatch_gather",
    )
    def k(x_hbm, pab_hbm, xs_hbm, pos_v, tok_v, rows0, rows1, s0, s1):
        wid = lax.axis_index("s") * 2 + lax.axis_index("c")
        base = wid * b_per_w
        pltpu.sync_copy(pab_hbm, pos_v)
        lane = lax.iota(jnp.int32, 16)

        def init(j, carry):
            tok_v[pl.ds(j * 16, 16)] = (base + j * 16 + lane) & (T - 1)
            return carry

        lax.fori_loop(0, b_per_w // 16, init, 0)

        def scan(i, carry):
            p16 = pos_v[pl.ds(i * 16, 16)]
            rel = p16 - base
            msk = (rel >= 0) & (rel < b_per_w)
            t16 = (i * 16 + lane) & (T - 1)
            plsc.store_scatter(tok_v, [jnp.where(msk, rel, 0)], t16, mask=msk)
            return carry

        lax.fori_loop(0, (2 * T) // 16, scan, 0)

        bufs, sems = (rows0, rows1), (s0, s1)
        cps = [None] * nch
        for j in range(nch):
            b = j % 2
            if j >= 2:
                cps[j - 2].wait()
                pltpu.sync_copy(bufs[b],
                                xs_hbm.at[pl.ds(base + (j - 2) * chunk, chunk)])
            cps[j] = pltpu.async_copy(
                x_hbm.at[tok_v.at[pl.ds(j * chunk, chunk)]], bufs[b], sems[b])
        for j in range(max(nch - 2, 0), nch):
            cps[j].wait()
            pltpu.sync_copy(bufs[j % 2],
                            xs_hbm.at[pl.ds(base + j * chunk, chunk)])

    return k(xf, pab)


def _sc_gather(table, idx, nrows, chunk):
    b_per_w = nrows // NW
    nch = b_per_w // chunk
    dm = table.shape[1]
    dt = table.dtype

    @functools.partial(
        pl.kernel,
        mesh=_sc_mesh(),
        out_type=jax.ShapeDtypeStruct((nrows, dm), dt),
        scratch_types=[
            pltpu.VMEM((b_per_w,), jnp.int32),
            pltpu.VMEM((chunk, dm), dt),
            pltpu.VMEM((chunk, dm), dt),
            pltpu.SemaphoreType.DMA,
            pltpu.SemaphoreType.DMA,
        ],
        name=f"sc_gather{nrows}",
    )
    def k(table_hbm, idx_hbm, out_hbm, idx_v, rows0, rows1, s0, s1):
        wid = lax.axis_index("s") * 2 + lax.axis_index("c")
        base = wid * b_per_w
        pltpu.sync_copy(idx_hbm.at[pl.ds(base, b_per_w)], idx_v)

        bufs, sems = (rows0, rows1), (s0, s1)
        cps = [None] * nch
        for j in range(nch):
            b = j % 2
            if j >= 2:
                cps[j - 2].wait()
                pltpu.sync_copy(bufs[b],
                                out_hbm.at[pl.ds(base + (j - 2) * chunk, chunk)])
            cps[j] = pltpu.async_copy(
                table_hbm.at[idx_v.at[pl.ds(j * chunk, chunk)]], bufs[b],
                sems[b])
        for j in range(max(nch - 2, 0), nch):
            cps[j].wait()
            pltpu.sync_copy(bufs[j % 2],
                            out_hbm.at[pl.ds(base + j * chunk, chunk)])

    return k(table, idx)


def _ffn_body(ept_ref, xs_ref, w1_ref, b1_ref, g1_ref, be1_ref, w2_ref,
              b2_ref, ys_ref):
    e = ept_ref[pl.program_id(0)]

    @pl.when(e >= 0)
    def _():
        h = jnp.dot(xs_ref[...].astype(jnp.bfloat16),
                    w1_ref[0].astype(jnp.bfloat16),
                    preferred_element_type=jnp.float32)
        h = h + b1_ref[0]
        mu = jnp.mean(h, axis=-1, keepdims=True)
        var = jnp.mean((h - mu) ** 2, axis=-1, keepdims=True)
        h = (h - mu) * lax.rsqrt(var + EPS) * g1_ref[0] + be1_ref[0]
        h = 0.5 * h * (1.0 + lax.erf(h * (2.0 ** -0.5)))
        y = jnp.dot(h.astype(jnp.bfloat16), w2_ref[0].astype(jnp.bfloat16),
                    preferred_element_type=jnp.float32)
        ys_ref[...] = y + b2_ref[0]


def _grouped_ffn(ept, xs, w1, b1, g1, be1, w2, b2):
    def clamp(ep, i):
        v = ep[i]
        return jnp.where(v < 0, E - 1, v)

    grid_spec = pltpu.PrefetchScalarGridSpec(
        num_scalar_prefetch=1,
        grid=(MAXT,),
        in_specs=[
            pl.BlockSpec((BT, D), lambda i, ep: (i, 0)),
            pl.BlockSpec((1, D, H), lambda i, ep: (clamp(ep, i), 0, 0)),
            pl.BlockSpec((1, 1, H), lambda i, ep: (clamp(ep, i), 0, 0)),
            pl.BlockSpec((1, 1, H), lambda i, ep: (clamp(ep, i), 0, 0)),
            pl.BlockSpec((1, 1, H), lambda i, ep: (clamp(ep, i), 0, 0)),
            pl.BlockSpec((1, H, D), lambda i, ep: (clamp(ep, i), 0, 0)),
            pl.BlockSpec((1, 1, D), lambda i, ep: (clamp(ep, i), 0, 0)),
        ],
        out_specs=pl.BlockSpec((BT, D), lambda i, ep: (i, 0)),
    )
    return pl.pallas_call(
        _ffn_body,
        grid_spec=grid_spec,
        out_shape=jax.ShapeDtypeStruct((P, D), jnp.float32),
    )(ept, xs, w1, b1.reshape(E, 1, H), g1.reshape(E, 1, H),
      be1.reshape(E, 1, H), w2, b2.reshape(E, 1, D))


def _combine_body(x_ref, ga_ref, gb_ref, wa_ref, wb_ref, gf_ref, bf_ref, o_ref):
    comb = (x_ref[...]
            + wa_ref[:, 0:1] * ga_ref[...]
            + wb_ref[:, 0:1] * gb_ref[...])
    mu = jnp.mean(comb, axis=-1, keepdims=True)
    var = jnp.mean((comb - mu) ** 2, axis=-1, keepdims=True)
    o_ref[...] = (comb - mu) * lax.rsqrt(var + EPS) * gf_ref[...] + bf_ref[...]


def _combine(xf, gath, wa, wb, gf, bf):
    nblk = T // BT
    return pl.pallas_call(
        _combine_body,
        grid=(nblk,),
        in_specs=[
            pl.BlockSpec((BT, D), lambda i: (i, 0)),
            pl.BlockSpec((BT, D), lambda i: (i, 0)),
            pl.BlockSpec((BT, D), lambda i: (i + nblk, 0)),
            pl.BlockSpec((BT, 128), lambda i: (i, 0)),
            pl.BlockSpec((BT, 128), lambda i: (i, 0)),
            pl.BlockSpec((1, D), lambda i: (0, 0)),
            pl.BlockSpec((1, D), lambda i: (0, 0)),
        ],
        out_specs=pl.BlockSpec((BT, D), lambda i: (i, 0)),
        out_shape=jax.ShapeDtypeStruct((T, D), jnp.float32),
    )(xf, gath, gath, wa, wb, gf, bf)


def kernel(x, Wg, W1, b1, g1, be1, W2, b2, gf, bf):
    orig_shape = x.shape
    xf = x.reshape(T, D)
    pab, wa, wb, ept = _router(xf, Wg)
    xs = _dispatch_gather(xf, pab)
    ys = _grouped_ffn(ept, xs, W1, b1, g1, be1, W2, b2)
    gath = _sc_gather(ys, pab, 2 * T, 64)
    out = _combine(xf, gath, wa, wb, gf.reshape(1, D), bf.reshape(1, D))
    return out.reshape(orig_shape)
